# baseline (speedup 1.0000x reference)
import functools

import jax
import jax.numpy as jnp
from jax import lax
from jax.experimental import pallas as pl
from jax.experimental.pallas import tpu as pltpu
from jax.experimental.pallas import tpu_sc as plsc

_B = 4096
_T = 8192
_L = 16
_D = 1024
_ROW = _L * _D

_BT = 2048
_TT = 1024
_EPS = 1e-8

_NC = 2
_NS = 16
_NW = _NC * _NS
_RPW = _B // _NW


def _argmax_body(q_ref, k_ref, idx_ref, max_sc, idx_sc):
    t = pl.program_id(1)
    nt = pl.num_programs(1)

    @pl.when(t == 0)
    def _init():
        max_sc[...] = jnp.full((_BT,), -jnp.inf, jnp.float32)
        idx_sc[...] = jnp.zeros((_BT,), jnp.int32)

    k = k_ref[...]
    kn = jnp.maximum(jnp.sqrt(jnp.sum(k * k, axis=1)), _EPS)
    dots = lax.dot_general(q_ref[...], k, (((1,), (1,)), ((), ())),
                           preferred_element_type=jnp.float32)
    sims = dots / kn[None, :]
    local_max = jnp.max(sims, axis=1)
    cols = lax.broadcasted_iota(jnp.int32, (_BT, _TT), 1)
    masked = jnp.where(sims == local_max[:, None], cols, _TT)
    local_idx = jnp.min(masked, axis=1) + t * _TT
    better = local_max > max_sc[...]
    max_sc[...] = jnp.where(better, local_max, max_sc[...])
    idx_sc[...] = jnp.where(better, local_idx, idx_sc[...])

    @pl.when(t == nt - 1)
    def _emit():
        idx_ref[...] = idx_sc[...]


def _compute_indices(query, prompt_keys):
    return pl.pallas_call(
        _argmax_body,
        grid=(_B // _BT, _T // _TT),
        in_specs=[
            pl.BlockSpec((_BT, _D), lambda b, t: (b, 0)),
            pl.BlockSpec((_TT, _D), lambda b, t: (t, 0)),
        ],
        out_specs=pl.BlockSpec((_BT,), lambda b, t: (b,)),
        out_shape=jax.ShapeDtypeStruct((_B,), jnp.int32),
        scratch_shapes=[
            pltpu.VMEM((_BT,), jnp.float32),
            pltpu.VMEM((_BT,), jnp.int32),
        ],
    )(query, prompt_keys)


_NBUF = 4


def _gather_body(table_hbm, idx_hbm, out_hbm, idx_v, bufs, gsems, ssems):
    w = lax.axis_index("c") * _NS + lax.axis_index("s")
    pltpu.sync_copy(idx_hbm.at[w], idx_v)
    base = w * _RPW

    def _start_gather(j, b):
        pltpu.async_copy(table_hbm.at[idx_v.at[j]], bufs[b], gsems[b])

    def _wait_gather(j, b):
        pltpu.make_async_copy(table_hbm.at[idx_v.at[j]], bufs[b],
                              gsems[b]).wait()

    def _start_scatter(j, b):
        pltpu.async_copy(bufs[b], out_hbm.at[pl.ds(base + j, 1)], ssems[b])

    def _wait_scatter(j, b):
        pltpu.make_async_copy(bufs[b], out_hbm.at[pl.ds(base + j, 1)],
                              ssems[b]).wait()

    for b in range(_NBUF):
        _start_gather(b, b)

    def _group(g, carry):
        for b in range(_NBUF):
            j = g * _NBUF + b
            _wait_gather(j, b)
            _start_scatter(j, b)

        @pl.when(g < _RPW // _NBUF - 1)
        def _refill():
            for b in range(_NBUF):
                j = g * _NBUF + b
                _wait_scatter(j, b)
                _start_gather(j + _NBUF, b)

        return carry

    lax.fori_loop(0, _RPW // _NBUF, _group, 0)
    for b in range(_NBUF):
        _wait_scatter(_RPW - _NBUF + b, b)


@functools.cache
def _make_gather_rows():
    return functools.partial(
        pl.kernel,
        out_type=jax.ShapeDtypeStruct((_B, _ROW), jnp.float32),
        mesh=plsc.VectorSubcoreMesh(core_axis_name="c", subcore_axis_name="s"),
        scratch_types=[
            pltpu.VMEM((_RPW, 1), jnp.int32),
            tuple(pltpu.VMEM((1, _ROW), jnp.float32) for _ in range(_NBUF)),
            tuple(pltpu.SemaphoreType.DMA for _ in range(_NBUF)),
            tuple(pltpu.SemaphoreType.DMA for _ in range(_NBUF)),
        ],
    )(_gather_body)


def _probe_stream4_body(table_hbm, idx_hbm, out_hbm, idx_v, buf, sem):
    w = lax.axis_index("c") * _NS + lax.axis_index("s")
    pltpu.sync_copy(idx_hbm.at[w], idx_v)
    base = w * _RPW

    def _chunk(j, carry):
        pltpu.async_copy(table_hbm.at[idx_v.at[j]], buf, sem).wait()
        return carry

    lax.fori_loop(0, 32, _chunk, 0)
    pltpu.async_copy(buf, out_hbm.at[pl.ds(base, 4)], sem).wait()


@functools.cache
def _make_probe_stream4():
    return functools.partial(
        pl.kernel,
        out_type=jax.ShapeDtypeStruct((_B, _ROW), jnp.float32),
        mesh=plsc.VectorSubcoreMesh(core_axis_name="c", subcore_axis_name="s"),
        scratch_types=[
            pltpu.VMEM((32, 4), jnp.int32),
            pltpu.VMEM((4, _ROW), jnp.float32),
            pltpu.SemaphoreType.DMA,
        ],
    )(_probe_stream4_body)


def kernel(query, prompts, prompt_keys):
    idx = (jnp.arange(_B, dtype=jnp.int32) * 2 + query[0, 0].astype(jnp.int32)) % _T
    table = prompts.reshape(_T, _ROW)
    out = _make_probe_stream4()(table, idx.reshape(_NW, 32, 4))
    return out.reshape(_B, _L, _D)


# speedup vs baseline: 1.4591x; 1.4591x over previous
import functools

import jax
import jax.numpy as jnp
from jax import lax
from jax.experimental import pallas as pl
from jax.experimental.pallas import tpu as pltpu

_B = 4096
_T = 8192
_L = 16
_D = 1024

_BT = 2048
_TT = 1024
_EPS = 1e-8

_R = 8
_NSLOT = 4
_NCHUNK = _B // _R


def _argmax_body(q_ref, k_ref, idx_ref, max_sc, idx_sc):
    t = pl.program_id(1)
    nt = pl.num_programs(1)

    @pl.when(t == 0)
    def _init():
        max_sc[...] = jnp.full((_BT,), -jnp.inf, jnp.float32)
        idx_sc[...] = jnp.zeros((_BT,), jnp.int32)

    k = k_ref[...]
    kn = jnp.maximum(jnp.sqrt(jnp.sum(k * k, axis=1)), _EPS)
    dots = lax.dot_general(q_ref[...], k, (((1,), (1,)), ((), ())),
                           preferred_element_type=jnp.float32)
    sims = dots / kn[None, :]
    local_max = jnp.max(sims, axis=1)
    cols = lax.broadcasted_iota(jnp.int32, (_BT, _TT), 1)
    masked = jnp.where(sims == local_max[:, None], cols, _TT)
    local_idx = jnp.min(masked, axis=1) + t * _TT
    better = local_max > max_sc[...]
    max_sc[...] = jnp.where(better, local_max, max_sc[...])
    idx_sc[...] = jnp.where(better, local_idx, idx_sc[...])

    @pl.when(t == nt - 1)
    def _emit():
        idx_ref[...] = idx_sc[...]


def _compute_indices(query, prompt_keys):
    return pl.pallas_call(
        _argmax_body,
        grid=(_B // _BT, _T // _TT),
        in_specs=[
            pl.BlockSpec((_BT, _D), lambda b, t: (b, 0)),
            pl.BlockSpec((_TT, _D), lambda b, t: (t, 0)),
        ],
        out_specs=pl.BlockSpec((_BT,), lambda b, t: (b,)),
        out_shape=jax.ShapeDtypeStruct((_B,), jnp.int32),
        scratch_shapes=[
            pltpu.VMEM((_BT,), jnp.float32),
            pltpu.VMEM((_BT,), jnp.int32),
        ],
    )(query, prompt_keys)


def _tc_gather_body(idx_ref, table, out, buf, gsem, ssem):
    def _start_gather(c, b):
        for r in range(_R):
            s = idx_ref[c * _R + r]
            pltpu.async_copy(table.at[pl.ds(s, 1)],
                             buf.at[b, pl.ds(r, 1)], gsem.at[b])

    def _wait_gather(c, b):
        for r in range(_R):
            s = idx_ref[c * _R + r]
            pltpu.make_async_copy(table.at[pl.ds(s, 1)],
                                  buf.at[b, pl.ds(r, 1)], gsem.at[b]).wait()

    def _start_scatter(c, b):
        pltpu.async_copy(buf.at[b], out.at[pl.ds(c * _R, _R)], ssem.at[b])

    def _wait_scatter(c, b):
        pltpu.make_async_copy(buf.at[b], out.at[pl.ds(c * _R, _R)],
                              ssem.at[b]).wait()

    for b in range(_NSLOT):
        _start_gather(b, b)

    def _group(g, carry):
        for b in range(_NSLOT):
            c = g * _NSLOT + b
            _wait_gather(c, b)
            _start_scatter(c, b)

        @pl.when(g < _NCHUNK // _NSLOT - 1)
        def _refill():
            for b in range(_NSLOT):
                c = g * _NSLOT + b
                _wait_scatter(c, b)
                _start_gather(c + _NSLOT, b)

        return carry

    lax.fori_loop(0, _NCHUNK // _NSLOT, _group, 0)
    for b in range(_NSLOT):
        _wait_scatter(_NCHUNK - _NSLOT + b, b)


def _tc_gather(prompts, idx):
    return pl.pallas_call(
        _tc_gather_body,
        grid_spec=pltpu.PrefetchScalarGridSpec(
            num_scalar_prefetch=1,
            grid=(1,),
            in_specs=[pl.BlockSpec(memory_space=pl.ANY)],
            out_specs=pl.BlockSpec(memory_space=pl.ANY),
            scratch_shapes=[
                pltpu.VMEM((_NSLOT, _R, _L, _D), jnp.float32),
                pltpu.SemaphoreType.DMA((_NSLOT,)),
                pltpu.SemaphoreType.DMA((_NSLOT,)),
            ],
        ),
        out_shape=jax.ShapeDtypeStruct((_B, _L, _D), jnp.float32),
    )(idx, prompts)


def kernel(query, prompts, prompt_keys):
    idx = _compute_indices(query, prompt_keys)
    return _tc_gather(prompts, idx)


# speedup vs baseline: 1.4750x; 1.0109x over previous
import functools

import jax
import jax.numpy as jnp
from jax import lax
from jax.experimental import pallas as pl
from jax.experimental.pallas import tpu as pltpu

_B = 4096
_T = 8192
_L = 16
_D = 1024

_BT = 2048
_TT = 1024
_NBC = _B // _BT
_NTT = _T // _TT
_EPS = 1e-8

_R = 8
_NSLOT = 4
_NCHUNK = _B // _R
_CPS = _BT // _R // _NTT


def _fused_body(q_ref, k_ref, table, out, max_sc, idx_sc, idx_smem,
                buf, gsem, ssem):
    c = pl.program_id(0)
    t = pl.program_id(1)
    s = c * _NTT + t

    @pl.when(c < _NBC)
    def _compute():
        @pl.when(t == 0)
        def _init():
            max_sc[...] = jnp.full((_BT,), -jnp.inf, jnp.float32)
            idx_sc[...] = jnp.zeros((_BT,), jnp.int32)

        k = k_ref[...]
        kn = jnp.maximum(jnp.sqrt(jnp.sum(k * k, axis=1)), _EPS)
        dots = lax.dot_general(q_ref[...], k, (((1,), (1,)), ((), ())),
                               preferred_element_type=jnp.float32)
        sims = dots / kn[None, :]
        local_max = jnp.max(sims, axis=1)
        cols = lax.broadcasted_iota(jnp.int32, (_BT, _TT), 1)
        masked = jnp.where(sims == local_max[:, None], cols, _TT)
        local_idx = jnp.min(masked, axis=1) + t * _TT
        better = local_max > max_sc[...]
        max_sc[...] = jnp.where(better, local_max, max_sc[...])
        idx_sc[...] = jnp.where(better, local_idx, idx_sc[...])

        @pl.when(t == _NTT - 1)
        def _publish():
            pltpu.sync_copy(idx_sc, idx_smem.at[pl.ds(c * _BT, _BT)])

    def _start_gather(kc, b):
        for r in range(_R):
            row = idx_smem[kc * _R + r]
            pltpu.async_copy(table.at[pl.ds(row, 1)],
                             buf.at[b, pl.ds(r, 1)], gsem.at[b])

    def _wait_gather(kc, b):
        for r in range(_R):
            row = idx_smem[kc * _R + r]
            pltpu.make_async_copy(table.at[pl.ds(row, 1)],
                                  buf.at[b, pl.ds(r, 1)], gsem.at[b]).wait()

    def _start_scatter(kc, b):
        pltpu.async_copy(buf.at[b], out.at[pl.ds(kc * _R, _R)], ssem.at[b])

    def _wait_scatter(kc, b):
        pltpu.make_async_copy(buf.at[b], out.at[pl.ds(kc * _R, _R)],
                              ssem.at[b]).wait()

    @pl.when(s >= _NTT)
    def _gather_phase():
        kbase = (s - _NTT) * _CPS

        @pl.when(s == _NTT)
        def _prime():
            for b in range(_NSLOT):
                _start_gather(b, b)

        def _group(g, carry):
            k0 = kbase + g * _NSLOT
            for b in range(_NSLOT):
                _wait_gather(k0 + b, b)
                _start_scatter(k0 + b, b)

            @pl.when(k0 < _NCHUNK - _NSLOT)
            def _refill():
                for b in range(_NSLOT):
                    _wait_scatter(k0 + b, b)
                    _start_gather(k0 + b + _NSLOT, b)

            return carry

        lax.fori_loop(0, _CPS // _NSLOT, _group, 0)

        @pl.when(s == (_NBC + 1) * _NTT - 1)
        def _drain():
            for b in range(_NSLOT):
                _wait_scatter(_NCHUNK - _NSLOT + b, b)


def kernel(query, prompts, prompt_keys):
    return pl.pallas_call(
        _fused_body,
        grid=(_NBC + 1, _NTT),
        in_specs=[
            pl.BlockSpec((_BT, _D),
                         lambda c, t: (jnp.minimum(c, _NBC - 1), 0)),
            pl.BlockSpec((_TT, _D),
                         lambda c, t: (jnp.where(c < _NBC, t, _NTT - 1), 0)),
            pl.BlockSpec(memory_space=pl.ANY),
        ],
        out_specs=pl.BlockSpec(memory_space=pl.ANY),
        out_shape=jax.ShapeDtypeStruct((_B, _L, _D), jnp.float32),
        scratch_shapes=[
            pltpu.VMEM((_BT,), jnp.float32),
            pltpu.VMEM((_BT,), jnp.int32),
            pltpu.SMEM((_B,), jnp.int32),
            pltpu.VMEM((_NSLOT, _R, _L, _D), jnp.float32),
            pltpu.SemaphoreType.DMA((_NSLOT,)),
            pltpu.SemaphoreType.DMA((_NSLOT,)),
        ],
    )(query, prompt_keys, prompts)


# speedup vs baseline: 1.9530x; 1.3240x over previous
import functools

import jax
import jax.numpy as jnp
from jax import lax
from jax.experimental import pallas as pl
from jax.experimental.pallas import tpu as pltpu

_B = 4096
_T = 8192
_L = 16
_D = 1024

_BT = 2048
_TT = 1024
_NBC = _B // _BT
_NTT = _T // _TT
_EPS = 1e-8

_R = 16
_NSLOT = 16
_NCHUNK = _B // _R
_CPS = _BT // _R // _NTT


def _fused_body(q_ref, k_ref, table, out, max_sc, idx_sc, idx_smem,
                buf, gsem, ssem):
    c = pl.program_id(0)
    t = pl.program_id(1)
    s = c * _NTT + t

    @pl.when(c < _NBC)
    def _compute():
        @pl.when(t == 0)
        def _init():
            max_sc[...] = jnp.full((_BT,), -jnp.inf, jnp.float32)
            idx_sc[...] = jnp.zeros((_BT,), jnp.int32)

        k = k_ref[...]
        kn = jnp.maximum(jnp.sqrt(jnp.sum(k * k, axis=1)), _EPS)
        dots = lax.dot_general(q_ref[...], k, (((1,), (1,)), ((), ())),
                               preferred_element_type=jnp.float32)
        sims = dots / kn[None, :]
        local_max = jnp.max(sims, axis=1)
        cols = lax.broadcasted_iota(jnp.int32, (_BT, _TT), 1)
        masked = jnp.where(sims == local_max[:, None], cols, _TT)
        local_idx = jnp.min(masked, axis=1) + t * _TT
        better = local_max > max_sc[...]
        max_sc[...] = jnp.where(better, local_max, max_sc[...])
        idx_sc[...] = jnp.where(better, local_idx, idx_sc[...])

        @pl.when(t == _NTT - 1)
        def _publish():
            pltpu.sync_copy(idx_sc, idx_smem.at[pl.ds(c * _BT, _BT)])

    def _start_gather(kc, b):
        for r in range(_R):
            row = idx_smem[kc * _R + r]
            pltpu.async_copy(table.at[pl.ds(row, 1)],
                             buf.at[b, pl.ds(r, 1)], gsem.at[b])

    def _wait_gather(kc, b):
        for r in range(_R):
            row = idx_smem[kc * _R + r]
            pltpu.make_async_copy(table.at[pl.ds(row, 1)],
                                  buf.at[b, pl.ds(r, 1)], gsem.at[b]).wait()

    def _start_scatter(kc, b):
        pltpu.async_copy(buf.at[b], out.at[pl.ds(kc * _R, _R)], ssem.at[b])

    def _wait_scatter(kc, b):
        pltpu.make_async_copy(buf.at[b], out.at[pl.ds(kc * _R, _R)],
                              ssem.at[b]).wait()

    @pl.when(s >= _NTT)
    def _gather_phase():
        kbase = (s - _NTT) * _CPS

        @pl.when(s == _NTT)
        def _prime():
            for b in range(_NSLOT):
                _start_gather(b, b)

        def _group(g, carry):
            k0 = kbase + g * _NSLOT
            for b in range(_NSLOT):
                _wait_gather(k0 + b, b)
                _start_scatter(k0 + b, b)

            @pl.when(k0 < _NCHUNK - _NSLOT)
            def _refill():
                for b in range(_NSLOT):
                    _wait_scatter(k0 + b, b)
                    _start_gather(k0 + b + _NSLOT, b)

            return carry

        lax.fori_loop(0, _CPS // _NSLOT, _group, 0)

        @pl.when(s == (_NBC + 1) * _NTT - 1)
        def _drain():
            for b in range(_NSLOT):
                _wait_scatter(_NCHUNK - _NSLOT + b, b)


def kernel(query, prompts, prompt_keys):
    return pl.pallas_call(
        _fused_body,
        grid=(_NBC + 1, _NTT),
        in_specs=[
            pl.BlockSpec((_BT, _D),
                         lambda c, t: (jnp.minimum(c, _NBC - 1), 0)),
            pl.BlockSpec((_TT, _D),
                         lambda c, t: (jnp.where(c < _NBC, t, _NTT - 1), 0)),
            pl.BlockSpec(memory_space=pl.ANY),
        ],
        out_specs=pl.BlockSpec(memory_space=pl.ANY),
        out_shape=jax.ShapeDtypeStruct((_B, _L, _D), jnp.float32),
        scratch_shapes=[
            pltpu.VMEM((_BT,), jnp.float32),
            pltpu.VMEM((_BT,), jnp.int32),
            pltpu.SMEM((_B,), jnp.int32),
            pltpu.VMEM((_NSLOT, _R, _L, _D), jnp.float32),
            pltpu.SemaphoreType.DMA((_NSLOT,)),
            pltpu.SemaphoreType.DMA((_NSLOT,)),
        ],
    )(query, prompt_keys, prompts)


# speedup vs baseline: 1.9559x; 1.0015x over previous
import jax
import jax.numpy as jnp
from jax import lax
from jax.experimental import pallas as pl
from jax.experimental.pallas import tpu as pltpu

_B = 4096
_T = 8192
_L = 16
_D = 1024

_BT = 2048
_TT = 1024
_NBC = _B // _BT
_NTT = _T // _TT
_EPS = 1e-8

_R = 16
_NSLOT = 16
_NCHUNK = _B // _R
_CPS = _BT // _R // _NTT


def _fused_body(q_ref, k_ref, table, out, max_sc, idx_sc, idx_smem,
                buf, gsem, ssem):
    c = pl.program_id(0)
    t = pl.program_id(1)
    s = c * _NTT + t

    @pl.when(c < _NBC)
    def _compute():
        @pl.when(t == 0)
        def _init():
            max_sc[...] = jnp.full((_BT,), -jnp.inf, jnp.float32)
            idx_sc[...] = jnp.zeros((_BT,), jnp.int32)

        k = k_ref[...]
        kn = jnp.maximum(jnp.sqrt(jnp.sum(k * k, axis=1)), _EPS)
        dots = lax.dot_general(q_ref[...], k, (((1,), (1,)), ((), ())),
                               preferred_element_type=jnp.float32)
        sims = dots / kn[None, :]
        local_max = jnp.max(sims, axis=1)
        cols = lax.broadcasted_iota(jnp.int32, (_BT, _TT), 1)
        masked = jnp.where(sims == local_max[:, None], cols, _TT)
        local_idx = jnp.min(masked, axis=1) + t * _TT
        better = local_max > max_sc[...]
        max_sc[...] = jnp.where(better, local_max, max_sc[...])
        idx_sc[...] = jnp.where(better, local_idx, idx_sc[...])

        @pl.when(t == _NTT - 1)
        def _publish():
            pltpu.sync_copy(idx_sc, idx_smem.at[pl.ds(c * _BT, _BT)])

    def _start_gather(kc, b):
        for r in range(_R):
            row = idx_smem[kc * _R + r]
            pltpu.async_copy(table.at[pl.ds(row, 1)],
                             buf.at[b, pl.ds(r, 1)], gsem.at[b])

    def _wait_gather(kc, b):
        for r in range(_R):
            row = idx_smem[kc * _R + r]
            pltpu.make_async_copy(table.at[pl.ds(row, 1)],
                                  buf.at[b, pl.ds(r, 1)], gsem.at[b]).wait()

    def _start_scatter(kc, b):
        pltpu.async_copy(buf.at[b], out.at[pl.ds(kc * _R, _R)], ssem.at[b])

    def _wait_scatter(kc, b):
        pltpu.make_async_copy(buf.at[b], out.at[pl.ds(kc * _R, _R)],
                              ssem.at[b]).wait()

    @pl.when(s >= _NTT)
    def _gather_phase():
        kbase = (s - _NTT) * _CPS

        @pl.when(s == _NTT)
        def _prime():
            for b in range(_NSLOT):
                _start_gather(b, b)

        def _group(g, carry):
            k0 = kbase + g * _NSLOT
            for b in range(_NSLOT):
                _wait_gather(k0 + b, b)
                _start_scatter(k0 + b, b)

            @pl.when(k0 < _NCHUNK - _NSLOT)
            def _refill():
                for b in range(_NSLOT):
                    _wait_scatter(k0 + b, b)
                    _start_gather(k0 + b + _NSLOT, b)

            return carry

        lax.fori_loop(0, _CPS // _NSLOT, _group, 0)

        @pl.when(s == (_NBC + 1) * _NTT - 1)
        def _drain():
            for b in range(_NSLOT):
                _wait_scatter(_NCHUNK - _NSLOT + b, b)


def kernel(query, prompts, prompt_keys):
    return pl.pallas_call(
        _fused_body,
        grid=(_NBC + 1, _NTT),
        in_specs=[
            pl.BlockSpec((_BT, _D),
                         lambda c, t: (jnp.minimum(c, _NBC - 1), 0)),
            pl.BlockSpec((_TT, _D),
                         lambda c, t: (jnp.where(c < _NBC, t, _NTT - 1), 0)),
            pl.BlockSpec(memory_space=pl.ANY),
        ],
        out_specs=pl.BlockSpec(memory_space=pl.ANY),
        out_shape=jax.ShapeDtypeStruct((_B, _L, _D), jnp.float32),
        scratch_shapes=[
            pltpu.VMEM((_BT,), jnp.float32),
            pltpu.VMEM((_BT,), jnp.int32),
            pltpu.SMEM((_B,), jnp.int32),
            pltpu.VMEM((_NSLOT, _R, _L, _D), jnp.float32),
            pltpu.SemaphoreType.DMA((_NSLOT,)),
            pltpu.SemaphoreType.DMA((_NSLOT,)),
        ],
    )(query, prompt_keys, prompts)
